# baseline (speedup 1.0000x reference)
import functools

import jax
import jax.numpy as jnp
from jax import lax
from jax.experimental import pallas as pl
from jax.experimental.pallas import tpu as pltpu
from jax.experimental.pallas import tpu_sc as plsc

N = 10000
NPAD = 10752
E = 320000
CH = 128
NCH = 80
NW = 32
EP = NW * NCH * CH
G = 16
F = 20

_MESH = plsc.VectorSubcoreMesh(
    core_axis_name="c", subcore_axis_name="s", num_cores=2, num_subcores=16)
_SC_PARAMS = pltpu.CompilerParams(use_tc_tiling_on_sc=False)


def _make_s_kernel(width, nbuf):
  ng = NCH // nbuf

  @functools.partial(
      pl.kernel,
      mesh=_MESH,
      compiler_params=_SC_PARAMS,
      out_type=jax.ShapeDtypeStruct((2, NPAD, width), jnp.float32),
      scratch_types=[
          pltpu.VMEM((NCH, CH), jnp.int32),
          pltpu.VMEM((NCH, CH), jnp.int32),
          pltpu.VMEM((112, width), jnp.float32),
          pltpu.VMEM_SHARED((NPAD, width), jnp.float32),
      ] + [pltpu.VMEM((CH, width), jnp.float32) for _ in range(nbuf)]
        + [pltpu.SemaphoreType.DMA for _ in range(2 * nbuf)],
  )
  def s_kernel(u_hbm, rowg_hbm, cols_hbm, z_hbm, out_hbm,
               rowv, colv, sbuf, acc, *bufs_sems):
    rbufs = bufs_sems[:nbuf]
    sg = bufs_sems[nbuf:2 * nbuf]
    ss = bufs_sems[2 * nbuf:]
    c = lax.axis_index("c")
    s = lax.axis_index("s")
    w = s * 2 + c
    pltpu.sync_copy(z_hbm.at[pl.ds(0, 112)], sbuf)
    for t in range(6):
      pltpu.sync_copy(sbuf, acc.at[pl.ds(s * 672 + t * 112, 112)])
    pltpu.sync_copy(rowg_hbm.at[w], rowv)
    pltpu.sync_copy(cols_hbm.at[w], colv)
    plsc.subcore_barrier()

    def g_start(j, i):
      pltpu.async_copy(u_hbm.at[rowv.at[j]], rbufs[i], sg[i])

    def g_wait(j, i):
      pltpu.make_async_copy(u_hbm.at[rowv.at[j]], rbufs[i], sg[i]).wait()

    def s_start(j, i):
      pltpu.async_copy(rbufs[i], acc.at[colv.at[j]], ss[i], add=True)

    def s_wait(j, i):
      pltpu.make_async_copy(rbufs[i], acc.at[colv.at[j]], ss[i]).wait()

    for i in range(nbuf):
      g_start(i, i)

    def outer(g, carry):
      for i in range(nbuf):
        j = g * nbuf + i
        g_wait(j, i)
        s_start(j, i)
      for i in range(nbuf):
        j = g * nbuf + i
        s_wait(j, i)
        g_start(j + nbuf, i)
      return carry

    lax.fori_loop(0, ng - 1, outer, 0)
    for i in range(nbuf):
      j = (ng - 1) * nbuf + i
      g_wait(j, i)
      s_start(j, i)
    for i in range(nbuf):
      j = (ng - 1) * nbuf + i
      s_wait(j, i)
    plsc.subcore_barrier()
    for t in range(6):
      pltpu.sync_copy(acc.at[pl.ds(s * 672 + t * 112, 112)], sbuf)
      pltpu.sync_copy(sbuf, out_hbm.at[c, pl.ds(s * 672 + t * 112, 112)])

  return s_kernel


@functools.partial(
    pl.kernel,
    mesh=_MESH,
    compiler_params=_SC_PARAMS,
    out_type=jax.ShapeDtypeStruct((2 * NPAD,), jnp.float32),
    scratch_types=[
        pltpu.VMEM((NCH, CH), jnp.int32),
        pltpu.VMEM((CH,), jnp.float32),
        pltpu.VMEM((672,), jnp.float32),
        pltpu.VMEM_SHARED((NPAD,), jnp.float32),
    ] + [pltpu.SemaphoreType.DMA for _ in range(8)],
)
def _deg_kernel(rows_hbm, z1_hbm, ones_hbm, out_hbm, rowv, ones_v, sbuf,
                acc, *sems):
  c = lax.axis_index("c")
  s = lax.axis_index("s")
  w = s * 2 + c
  pltpu.sync_copy(z1_hbm.at[pl.ds(s * 672, 672)], sbuf)
  pltpu.sync_copy(sbuf, acc.at[pl.ds(s * 672, 672)])
  pltpu.sync_copy(rows_hbm.at[w], rowv)
  pltpu.sync_copy(ones_hbm, ones_v)
  plsc.subcore_barrier()

  def body(g, carry):
    for i in range(8):
      pltpu.async_copy(ones_v, acc.at[rowv.at[g * 8 + i]], sems[i], add=True)
    for i in range(8):
      pltpu.make_async_copy(
          ones_v, acc.at[rowv.at[g * 8 + i]], sems[i]).wait()
    return carry

  lax.fori_loop(0, NCH // 8, body, 0)
  plsc.subcore_barrier()
  pltpu.sync_copy(acc.at[pl.ds(s * 672, 672)], sbuf)
  pltpu.sync_copy(sbuf, out_hbm.at[pl.ds(c * NPAD + s * 672, 672)])


def _bn_relu(h, g, be):
  m = jnp.mean(h, axis=0)
  v = jnp.mean((h - m) ** 2, axis=0)
  return jax.nn.relu((h - m) / jnp.sqrt(v + 1e-5) * g + be)


def _tc0_body(x_ref, w0_ref, p_ref, q_ref, a0_ref):
  x = x_ref[...]
  q = jnp.dot(x, w0_ref[2], preferred_element_type=jnp.float32)
  p_ref[...] = jnp.dot(x, w0_ref[1], preferred_element_type=jnp.float32)
  q_ref[...] = q
  a0_ref[...] = jnp.dot(x, w0_ref[0], preferred_element_type=jnp.float32) - q


def _tc1_body(degp_ref, p_ref, q_ref, dis_ref, u12_ref):
  deg = degp_ref[0:N] + degp_ref[NPAD:NPAD + N]
  dis = jnp.where(deg > 0, lax.rsqrt(deg), 0.0)
  dis_ref[...] = dis
  d = dis[:, None]
  u12_ref[0:N, 0:F] = d * p_ref[...]
  u12_ref[0:N, 24:24 + F] = d * q_ref[...]


def _tc2_body(sab_ref, dis_ref, a0_ref, b0_ref, u3_ref, pre0_ref):
  s_a = sab_ref[0, :N, 0:F] + sab_ref[1, :N, 0:F]
  s_b = sab_ref[0, :N, 24:24 + F] + sab_ref[1, :N, 24:24 + F]
  dis = dis_ref[...]
  d = dis[:, None]
  u3_ref[0:N, 0:F] = d * d * s_b
  pre0_ref[...] = a0_ref[...] - d * s_a + b0_ref[...][None, :]


def _tc3_body(scp_ref, pre0_ref, dis_ref, g0_ref, be0_ref, lw_ref,
              h_ref, u_ref, emb_ref):
  s_c = scp_ref[0, :N, 0:F] + scp_ref[1, :N, 0:F]
  dis = dis_ref[...]
  d = dis[:, None]
  out0 = pre0_ref[...] + 2.0 * d * s_c
  h = _bn_relu(out0, g0_ref[...], be0_ref[...])
  h_ref[...] = h
  u_ref[0:N, 0:F] = d * h
  w = jax.nn.softmax(lw_ref[...])
  emb_ref[...] = w[0] * h


def _tca_body(s1p_ref, dis_ref, h_ref, wk_ref, bk_ref, u2_ref, pre_ref):
  s1 = s1p_ref[0, :N, 0:F] + s1p_ref[1, :N, 0:F]
  dis = dis_ref[...]
  d = dis[:, None]
  u2_ref[0:N, 0:F] = d * d * s1
  h = h_ref[...]
  pre_ref[...] = (
      jnp.dot(h, wk_ref[0] - wk_ref[2], preferred_element_type=jnp.float32)
      - jnp.dot(d * s1, wk_ref[1], preferred_element_type=jnp.float32)
      + bk_ref[...][None, :])


def _make_tcb_body(k):
  def _tcb_body(s2p_ref, pre_ref, dis_ref, hin_ref, wk_ref, gk_ref, bek_ref,
                emb_ref, lw_ref, h_ref, u_ref, embo_ref):
    s2 = s2p_ref[0, :N, 0:F] + s2p_ref[1, :N, 0:F]
    dis = dis_ref[...]
    d = dis[:, None]
    out = pre_ref[...] + 2.0 * jnp.dot(
        d * s2, wk_ref[2], preferred_element_type=jnp.float32)
    h = _bn_relu(out, gk_ref[...], bek_ref[...]) + 0.7 * hin_ref[...]
    h_ref[...] = h
    u_ref[0:N, 0:F] = d * h
    w = jax.nn.softmax(lw_ref[...])
    embo_ref[...] = emb_ref[...] + w[k] * h
  return _tcb_body


def _tcf_body(s2p_ref, pre_ref, dis_ref, hin_ref, wk_ref, gk_ref, bek_ref,
              emb_ref, lw_ref, batch_ref, fcw_ref, fcb_ref, out_ref):
  s2 = s2p_ref[0, :N, 0:F] + s2p_ref[1, :N, 0:F]
  dis = dis_ref[...]
  d = dis[:, None]
  out = pre_ref[...] + 2.0 * jnp.dot(
      d * s2, wk_ref[2], preferred_element_type=jnp.float32)
  h = _bn_relu(out, gk_ref[...], bek_ref[...]) + 0.7 * hin_ref[...]
  w = jax.nn.softmax(lw_ref[...])
  emb = emb_ref[...] + w[3] * h
  b = batch_ref[...]
  onehot = jnp.where(
      b[None, :] == lax.broadcasted_iota(jnp.int32, (G, N), 0), 1.0, 0.0)
  sums = jnp.dot(onehot, emb, preferred_element_type=jnp.float32)
  cnt = jnp.sum(onehot, axis=1)
  pooled = sums / jnp.maximum(cnt, 1.0)[:, None]
  out_ref[...] = (
      jnp.dot(pooled, fcw_ref[...], preferred_element_type=jnp.float32)
      + fcb_ref[...][None, :])


def _tc_call(body, out_shapes):
  return pl.pallas_call(
      body,
      out_shape=out_shapes,
  )


_f32 = jnp.float32
_S32 = _make_s_kernel(32, 8)
_S48 = _make_s_kernel(48, 6)
_USHAPE = jax.ShapeDtypeStruct((NPAD, 32), _f32)
_TC0 = _tc_call(_tc0_body, (jax.ShapeDtypeStruct((N, F), _f32),
                            jax.ShapeDtypeStruct((N, F), _f32),
                            jax.ShapeDtypeStruct((N, F), _f32)))
_TC1 = _tc_call(_tc1_body, (jax.ShapeDtypeStruct((N,), _f32),
                            jax.ShapeDtypeStruct((NPAD, 48), _f32)))
_TC2 = _tc_call(_tc2_body, (_USHAPE,
                            jax.ShapeDtypeStruct((N, F), _f32)))
_TC3 = _tc_call(_tc3_body, (jax.ShapeDtypeStruct((N, F), _f32),
                            _USHAPE,
                            jax.ShapeDtypeStruct((N, F), _f32)))
_TCA = _tc_call(_tca_body, (_USHAPE,
                            jax.ShapeDtypeStruct((N, F), _f32)))
_TCB = {
    k: _tc_call(_make_tcb_body(k), (jax.ShapeDtypeStruct((N, F), _f32),
                                    _USHAPE,
                                    jax.ShapeDtypeStruct((N, F), _f32)))
    for k in (1, 2)
}
_TCF = _tc_call(_tcf_body, jax.ShapeDtypeStruct((G, 2), _f32))


def kernel(x, edge_index, batch, W0, b0, g0, be0, W1, b1, g1, be1,
           W2, b2, g2, be2, W3, b3, g3, be3, lw, fcW, fcb):
  row = edge_index[0]
  col = edge_index[1]
  npad_e = EP - E
  eid = jnp.arange(EP, dtype=jnp.int32)
  is_pad = eid >= E
  spread_g = eid % N
  spread_s = N + eid % (NPAD - N)
  row_p = jnp.pad(row, (0, npad_e))
  col_p = jnp.pad(col, (0, npad_e))
  row_g = jnp.where(is_pad, spread_g, row_p).reshape(NW, NCH, CH)
  col_s = jnp.where(is_pad, spread_s, col_p).reshape(NW, NCH, CH)
  row_s = jnp.where(is_pad, spread_s, row_p).reshape(NW, NCH, CH)
  z1 = jnp.zeros((NPAD,), _f32)
  z32 = jnp.zeros((112, 32), _f32)
  z48 = jnp.zeros((112, 48), _f32)
  ones = jnp.ones((CH,), _f32)

  degp = _deg_kernel(row_s, z1, ones)
  p0, q0, a0 = _TC0(x, W0)
  dis, u12 = _TC1(degp, p0, q0)
  sab = _S48(u12, row_g, col_s, z48)
  u3, pre0 = _TC2(sab, dis, a0, b0)
  scp = _S32(u3, row_g, col_s, z32)
  h, u, emb = _TC3(scp, pre0, dis, g0, be0, lw)

  for k, (Wk, bk, gk, bek) in enumerate(
      [(W1, b1, g1, be1), (W2, b2, g2, be2), (W3, b3, g3, be3)], start=1):
    s1p = _S32(u, row_g, col_s, z32)
    u2, pre = _TCA(s1p, dis, h, Wk, bk)
    s2p = _S32(u2, row_g, col_s, z32)
    if k < 3:
      h, u, emb = _TCB[k](s2p, pre, dis, h, Wk, gk, bek, emb, lw)
    else:
      return _TCF(s2p, pre, dis, h, Wk, gk, bek, emb, lw, batch, fcW, fcb)


# speedup vs baseline: 1.0174x; 1.0174x over previous
import functools

import jax
import jax.numpy as jnp
from jax import lax
from jax.experimental import pallas as pl
from jax.experimental.pallas import tpu as pltpu
from jax.experimental.pallas import tpu_sc as plsc

N = 10000
NPAD = 10752
E = 320000
CH = 128
NCH = 80
NW = 32
EP = NW * NCH * CH
G = 16
F = 20

_MESH = plsc.VectorSubcoreMesh(
    core_axis_name="c", subcore_axis_name="s", num_cores=2, num_subcores=16)
_SC_PARAMS = pltpu.CompilerParams(use_tc_tiling_on_sc=False)


def _make_s_kernel(width, nbuf):
  ng = NCH // nbuf

  @functools.partial(
      pl.kernel,
      mesh=_MESH,
      compiler_params=_SC_PARAMS,
      out_type=jax.ShapeDtypeStruct((2, NPAD, width), jnp.float32),
      scratch_types=[
          pltpu.VMEM((NCH, CH), jnp.int32),
          pltpu.VMEM((NCH, CH), jnp.int32),
          pltpu.VMEM((112, width), jnp.float32),
          pltpu.VMEM_SHARED((NPAD, width), jnp.float32),
      ] + [pltpu.VMEM((CH, width), jnp.float32) for _ in range(nbuf)]
        + [pltpu.SemaphoreType.DMA for _ in range(2 * nbuf)],
  )
  def s_kernel(u_hbm, rowg_hbm, cols_hbm, z_hbm, out_hbm,
               rowv, colv, sbuf, acc, *bufs_sems):
    rbufs = bufs_sems[:nbuf]
    sg = bufs_sems[nbuf:2 * nbuf]
    ss = bufs_sems[2 * nbuf:]
    c = lax.axis_index("c")
    s = lax.axis_index("s")
    w = s * 2 + c
    pltpu.sync_copy(z_hbm.at[pl.ds(0, 112)], sbuf)
    for t in range(6):
      pltpu.sync_copy(sbuf, acc.at[pl.ds(s * 672 + t * 112, 112)])
    pltpu.sync_copy(rowg_hbm.at[w], rowv)
    pltpu.sync_copy(cols_hbm.at[w], colv)
    plsc.subcore_barrier()

    def g_start(j, i):
      pltpu.async_copy(u_hbm.at[rowv.at[j]], rbufs[i], sg[i])

    def g_wait(j, i):
      pltpu.make_async_copy(u_hbm.at[rowv.at[j]], rbufs[i], sg[i]).wait()

    def s_start(j, i):
      pltpu.async_copy(rbufs[i], acc.at[colv.at[j]], ss[i], add=True)

    def s_wait(j, i):
      pltpu.make_async_copy(rbufs[i], acc.at[colv.at[j]], ss[i]).wait()

    for i in range(nbuf):
      g_start(i, i)

    def outer(g, carry):
      for i in range(nbuf):
        j = g * nbuf + i
        g_wait(j, i)
        s_start(j, i)
      for i in range(nbuf):
        j = g * nbuf + i
        s_wait(j, i)
        g_start(j + nbuf, i)
      return carry

    lax.fori_loop(0, ng - 1, outer, 0)
    for i in range(nbuf):
      j = (ng - 1) * nbuf + i
      g_wait(j, i)
      s_start(j, i)
    for i in range(nbuf):
      j = (ng - 1) * nbuf + i
      s_wait(j, i)
    plsc.subcore_barrier()
    for t in range(6):
      pltpu.sync_copy(acc.at[pl.ds(s * 672 + t * 112, 112)], sbuf)
      pltpu.sync_copy(sbuf, out_hbm.at[c, pl.ds(s * 672 + t * 112, 112)])

  return s_kernel


@functools.partial(
    pl.kernel,
    mesh=_MESH,
    compiler_params=_SC_PARAMS,
    out_type=jax.ShapeDtypeStruct((2 * NPAD,), jnp.float32),
    scratch_types=[
        pltpu.VMEM((NCH, CH), jnp.int32),
        pltpu.VMEM((CH,), jnp.float32),
        pltpu.VMEM((672,), jnp.float32),
        pltpu.VMEM_SHARED((NPAD,), jnp.float32),
    ] + [pltpu.SemaphoreType.DMA for _ in range(8)],
)
def _deg_kernel(rows_hbm, z1_hbm, ones_hbm, out_hbm, rowv, ones_v, sbuf,
                acc, *sems):
  c = lax.axis_index("c")
  s = lax.axis_index("s")
  w = s * 2 + c
  pltpu.sync_copy(z1_hbm.at[pl.ds(s * 672, 672)], sbuf)
  pltpu.sync_copy(sbuf, acc.at[pl.ds(s * 672, 672)])
  pltpu.sync_copy(rows_hbm.at[w], rowv)
  pltpu.sync_copy(ones_hbm, ones_v)
  plsc.subcore_barrier()

  def body(g, carry):
    for i in range(8):
      pltpu.async_copy(ones_v, acc.at[rowv.at[g * 8 + i]], sems[i], add=True)
    for i in range(8):
      pltpu.make_async_copy(
          ones_v, acc.at[rowv.at[g * 8 + i]], sems[i]).wait()
    return carry

  lax.fori_loop(0, NCH // 8, body, 0)
  plsc.subcore_barrier()
  pltpu.sync_copy(acc.at[pl.ds(s * 672, 672)], sbuf)
  pltpu.sync_copy(sbuf, out_hbm.at[pl.ds(c * NPAD + s * 672, 672)])


def _bn_relu(h, g, be):
  m = jnp.mean(h, axis=0)
  v = jnp.mean((h - m) ** 2, axis=0)
  return jax.nn.relu((h - m) / jnp.sqrt(v + 1e-5) * g + be)


def _tc1_body(degp_ref, x_ref, w0_ref, dis_ref, u12_ref, a0_ref):
  deg = degp_ref[0:N] + degp_ref[NPAD:NPAD + N]
  dis = jnp.where(deg > 0, lax.rsqrt(deg), 0.0)
  dis_ref[...] = dis
  x = x_ref[...]
  p = jnp.dot(x, w0_ref[1], preferred_element_type=jnp.float32)
  q = jnp.dot(x, w0_ref[2], preferred_element_type=jnp.float32)
  a0_ref[...] = jnp.dot(x, w0_ref[0], preferred_element_type=jnp.float32) - q
  d = dis[:, None]
  u12_ref[0:N, 0:F] = d * p
  u12_ref[0:N, 24:24 + F] = d * q


def _tc2_body(sab_ref, dis_ref, a0_ref, b0_ref, u3_ref, pre0_ref):
  s_a = sab_ref[0, :N, 0:F] + sab_ref[1, :N, 0:F]
  s_b = sab_ref[0, :N, 24:24 + F] + sab_ref[1, :N, 24:24 + F]
  dis = dis_ref[...]
  d = dis[:, None]
  u3_ref[0:N, 0:F] = d * d * s_b
  pre0_ref[...] = a0_ref[...] - d * s_a + b0_ref[...][None, :]


def _tc3_body(scp_ref, pre0_ref, dis_ref, g0_ref, be0_ref, lw_ref,
              h_ref, u_ref, emb_ref):
  s_c = scp_ref[0, :N, 0:F] + scp_ref[1, :N, 0:F]
  dis = dis_ref[...]
  d = dis[:, None]
  out0 = pre0_ref[...] + 2.0 * d * s_c
  h = _bn_relu(out0, g0_ref[...], be0_ref[...])
  h_ref[...] = h
  u_ref[0:N, 0:F] = d * h
  w = jax.nn.softmax(lw_ref[...])
  emb_ref[...] = w[0] * h


def _tca_body(s1p_ref, dis_ref, h_ref, wk_ref, bk_ref, u2_ref, pre_ref):
  s1 = s1p_ref[0, :N, 0:F] + s1p_ref[1, :N, 0:F]
  dis = dis_ref[...]
  d = dis[:, None]
  u2_ref[0:N, 0:F] = d * d * s1
  h = h_ref[...]
  pre_ref[...] = (
      jnp.dot(h, wk_ref[0] - wk_ref[2], preferred_element_type=jnp.float32)
      - jnp.dot(d * s1, wk_ref[1], preferred_element_type=jnp.float32)
      + bk_ref[...][None, :])


def _make_tcb_body(k):
  def _tcb_body(s2p_ref, pre_ref, dis_ref, hin_ref, wk_ref, gk_ref, bek_ref,
                emb_ref, lw_ref, h_ref, u_ref, embo_ref):
    s2 = s2p_ref[0, :N, 0:F] + s2p_ref[1, :N, 0:F]
    dis = dis_ref[...]
    d = dis[:, None]
    out = pre_ref[...] + 2.0 * jnp.dot(
        d * s2, wk_ref[2], preferred_element_type=jnp.float32)
    h = _bn_relu(out, gk_ref[...], bek_ref[...]) + 0.7 * hin_ref[...]
    h_ref[...] = h
    u_ref[0:N, 0:F] = d * h
    w = jax.nn.softmax(lw_ref[...])
    embo_ref[...] = emb_ref[...] + w[k] * h
  return _tcb_body


def _tcf_body(s2p_ref, pre_ref, dis_ref, hin_ref, wk_ref, gk_ref, bek_ref,
              emb_ref, lw_ref, batch_ref, fcw_ref, fcb_ref, out_ref):
  s2 = s2p_ref[0, :N, 0:F] + s2p_ref[1, :N, 0:F]
  dis = dis_ref[...]
  d = dis[:, None]
  out = pre_ref[...] + 2.0 * jnp.dot(
      d * s2, wk_ref[2], preferred_element_type=jnp.float32)
  h = _bn_relu(out, gk_ref[...], bek_ref[...]) + 0.7 * hin_ref[...]
  w = jax.nn.softmax(lw_ref[...])
  emb = emb_ref[...] + w[3] * h
  b = batch_ref[...]
  onehot = jnp.where(
      b[None, :] == lax.broadcasted_iota(jnp.int32, (G, N), 0), 1.0, 0.0)
  sums = jnp.dot(onehot, emb, preferred_element_type=jnp.float32)
  cnt = jnp.sum(onehot, axis=1)
  pooled = sums / jnp.maximum(cnt, 1.0)[:, None]
  out_ref[...] = (
      jnp.dot(pooled, fcw_ref[...], preferred_element_type=jnp.float32)
      + fcb_ref[...][None, :])


def _tc_call(body, out_shapes):
  return pl.pallas_call(
      body,
      out_shape=out_shapes,
  )


_f32 = jnp.float32
_S32 = _make_s_kernel(32, 8)
_S48 = _make_s_kernel(48, 6)
_USHAPE = jax.ShapeDtypeStruct((NPAD, 32), _f32)
_TC1 = _tc_call(_tc1_body, (jax.ShapeDtypeStruct((N,), _f32),
                            jax.ShapeDtypeStruct((NPAD, 48), _f32),
                            jax.ShapeDtypeStruct((N, F), _f32)))
_TC2 = _tc_call(_tc2_body, (_USHAPE,
                            jax.ShapeDtypeStruct((N, F), _f32)))
_TC3 = _tc_call(_tc3_body, (jax.ShapeDtypeStruct((N, F), _f32),
                            _USHAPE,
                            jax.ShapeDtypeStruct((N, F), _f32)))
_TCA = _tc_call(_tca_body, (_USHAPE,
                            jax.ShapeDtypeStruct((N, F), _f32)))
_TCB = {
    k: _tc_call(_make_tcb_body(k), (jax.ShapeDtypeStruct((N, F), _f32),
                                    _USHAPE,
                                    jax.ShapeDtypeStruct((N, F), _f32)))
    for k in (1, 2)
}
_TCF = _tc_call(_tcf_body, jax.ShapeDtypeStruct((G, 2), _f32))


def kernel(x, edge_index, batch, W0, b0, g0, be0, W1, b1, g1, be1,
           W2, b2, g2, be2, W3, b3, g3, be3, lw, fcW, fcb):
  row = edge_index[0]
  col = edge_index[1]
  npad_e = EP - E
  eid = jnp.arange(EP, dtype=jnp.int32)
  is_pad = eid >= E
  spread_g = eid % N
  spread_s = N + eid % (NPAD - N)
  row_p = jnp.pad(row, (0, npad_e))
  col_p = jnp.pad(col, (0, npad_e))
  row_g = jnp.where(is_pad, spread_g, row_p).reshape(NW, NCH, CH)
  col_s = jnp.where(is_pad, spread_s, col_p).reshape(NW, NCH, CH)
  row_s = jnp.where(is_pad, spread_s, row_p).reshape(NW, NCH, CH)
  z1 = jnp.zeros((NPAD,), _f32)
  z32 = jnp.zeros((112, 32), _f32)
  z48 = jnp.zeros((112, 48), _f32)
  ones = jnp.ones((CH,), _f32)

  degp = _deg_kernel(row_s, z1, ones)
  dis, u12, a0 = _TC1(degp, x, W0)
  sab = _S48(u12, row_g, col_s, z48)
  u3, pre0 = _TC2(sab, dis, a0, b0)
  scp = _S32(u3, row_g, col_s, z32)
  h, u, emb = _TC3(scp, pre0, dis, g0, be0, lw)

  for k, (Wk, bk, gk, bek) in enumerate(
      [(W1, b1, g1, be1), (W2, b2, g2, be2), (W3, b3, g3, be3)], start=1):
    s1p = _S32(u, row_g, col_s, z32)
    u2, pre = _TCA(s1p, dis, h, Wk, bk)
    s2p = _S32(u2, row_g, col_s, z32)
    if k < 3:
      h, u, emb = _TCB[k](s2p, pre, dis, h, Wk, gk, bek, emb, lw)
    else:
      return _TCF(s2p, pre, dis, h, Wk, gk, bek, emb, lw, batch, fcW, fcb)
